# Initial kernel scaffold; baseline (speedup 1.0000x reference)
import jax
import jax.numpy as jnp
from jax.experimental import pallas as pl


def kernel(features, proposals, image_shapes, gt_boxes, gt_labels):
    raise NotImplementedError("write your pallas kernel here")


# speedup vs baseline: 2.5656x; 2.5656x over previous
import functools

import jax
import jax.numpy as jnp
from jax import lax
from jax.experimental import pallas as pl
from jax.experimental.pallas import tpu as pltpu
from jax.experimental.pallas import tpu_sc as plsc

_N_POS = 128
_N_NEG = 384
_S = _N_POS + _N_NEG


def _match_body(G, p_ref, gt_ref, mv_ref, mi_ref):
    x1 = p_ref[0, 0]
    y1 = p_ref[0, 1]
    x2 = p_ref[0, 2]
    y2 = p_ref[0, 3]
    area_b = (x2 - x1) * (y2 - y1)
    best = jnp.full(x1.shape, -1.0, jnp.float32)
    bidx = jnp.zeros(x1.shape, jnp.int32)
    for g in range(G):
        ax1 = gt_ref[0, 0, g]
        ay1 = gt_ref[0, 1, g]
        ax2 = gt_ref[0, 2, g]
        ay2 = gt_ref[0, 3, g]
        area_a = (ax2 - ax1) * (ay2 - ay1)
        w = jnp.maximum(jnp.minimum(ax2, x2) - jnp.maximum(ax1, x1), 0.0)
        h = jnp.maximum(jnp.minimum(ay2, y2) - jnp.maximum(ay1, y1), 0.0)
        inter = w * h
        iou = inter / (area_a + area_b - inter + 1e-9)
        upd = iou > best
        best = jnp.where(upd, iou, best)
        bidx = jnp.where(upd, g, bidx)
    mv_ref[0] = best
    mi_ref[0] = bidx


def _enc_body(sp_ref, gt_ref, out_ref):
    sp = sp_ref[0]
    g = gt_ref[0]
    px1, py1, px2, py2 = sp[0:1], sp[1:2], sp[2:3], sp[3:4]
    gx1, gy1, gx2, gy2 = g[0:1], g[1:2], g[2:3], g[3:4]
    pw = px2 - px1
    ph = py2 - py1
    pxc = px1 + 0.5 * pw
    pyc = py1 + 0.5 * ph
    gw = gx2 - gx1
    gh = gy2 - gy1
    gxc = gx1 + 0.5 * gw
    gyc = gy1 + 0.5 * gh
    dx = 10.0 * (gxc - pxc) / pw
    dy = 10.0 * (gyc - pyc) / ph
    dw = 5.0 * jnp.log(gw / pw)
    dh = 5.0 * jnp.log(gh / ph)
    out_ref[0] = jnp.concatenate([dx, dy, dw, dh], axis=0)


def kernel(features, proposals, image_shapes, gt_boxes, gt_labels):
    del features, image_shapes
    i32 = jnp.int32
    f32 = jnp.float32
    B, N, _ = proposals.shape
    G = gt_boxes.shape[1]
    NP = N + G
    NPAD = -(-NP // 128) * 128
    NCH = -(-NP // 16)

    props_t = jnp.transpose(
        jnp.concatenate([proposals, gt_boxes], axis=1), (0, 2, 1)
    ).astype(f32)
    props_tp = jnp.pad(props_t, ((0, 0), (0, 0), (0, NPAD - NP)))
    gt_t = jnp.transpose(gt_boxes, (0, 2, 1)).astype(f32)

    perm = jnp.stack([
        jnp.argsort(
            -jax.random.uniform(jax.random.fold_in(jax.random.key(42), i), (NP,)),
            stable=True,
        ).astype(i32)
        for i in range(B)
    ])
    perm_p = jnp.pad(perm, ((0, 0), (0, NPAD - NP)))

    R = 8
    C = NPAD // R
    mv, mi = pl.pallas_call(
        functools.partial(_match_body, G),
        grid=(B,),
        in_specs=[
            pl.BlockSpec((1, 4, R, C), lambda i: (i, 0, 0, 0)),
            pl.BlockSpec((1, 4, G), lambda i: (i, 0, 0), memory_space=pltpu.SMEM),
        ],
        out_specs=[
            pl.BlockSpec((1, R, C), lambda i: (i, 0, 0)),
            pl.BlockSpec((1, R, C), lambda i: (i, 0, 0)),
        ],
        out_shape=[
            jax.ShapeDtypeStruct((B, R, C), f32),
            jax.ShapeDtypeStruct((B, R, C), i32),
        ],
    )(props_tp.reshape(B, 4, R, C), gt_t)
    mv = mv.reshape(B, NPAD)
    mi = mi.reshape(B, NPAD)

    info = plsc.get_sparse_core_info()
    NC = info.num_cores
    mesh = plsc.VectorSubcoreMesh(core_axis_name="c", subcore_axis_name="s")

    def _sc_body(mv_h, mi_h, pe_h, pr_h, gt_h, gl_h, osp_h, ogt_h, olb_h,
                 mv_v, mi_v, pe_v, p0, p1, p2, p3, g0, g1, g2, g3, gl_v,
                 sa_v, o0, o1, o2, o3, q0, q1, q2, q3, ol_v):
        wid = lax.axis_index("s") * NC + lax.axis_index("c")

        @pl.when(wid < B)
        def _():
            img = wid
            pltpu.sync_copy(mv_h.at[img], mv_v)
            pltpu.sync_copy(mi_h.at[img], mi_v)
            pltpu.sync_copy(pe_h.at[img], pe_v)
            for c, pv in enumerate((p0, p1, p2, p3)):
                pltpu.sync_copy(pr_h.at[img, c], pv)
            for c, gv in enumerate((g0, g1, g2, g3)):
                pltpu.sync_copy(gt_h.at[img, c], gv)
            pltpu.sync_copy(gl_h.at[img], gl_v)

            iota16 = lax.iota(i32, 16)

            def pass1(t, carry):
                cp, cn = carry
                base = t * 16
                idx = pe_v[pl.ds(base, 16)]
                valid = (base + iota16) < NP
                hit = plsc.load_gather(mv_v, [idx]) >= 0.5
                posm = hit & valid
                negm = (~hit) & valid
                pi = posm.astype(i32)
                ni = negm.astype(i32)
                gp = cp + plsc.cumsum(pi) - pi
                gn = cn + plsc.cumsum(ni) - ni
                plsc.store_scatter(sa_v, [gp], idx, mask=posm & (gp < _N_POS))
                plsc.store_scatter(sa_v, [gn + _N_POS], idx,
                                   mask=negm & (gn < _N_NEG))
                return cp + jnp.sum(pi), cn + jnp.sum(ni)

            tp, tn = lax.fori_loop(0, NCH, pass1, (i32(0), i32(0)))

            def fill_body(t, carry):
                cz, cf = carry
                base = t * 16
                lanes = base + iota16
                valid = lanes < NP
                hit = mv_v[pl.ds(base, 16)] >= 0.5
                zm = (~hit) & valid
                pm = hit & valid
                zi = zm.astype(i32)
                fi = pm.astype(i32)
                s1 = tp + cz + plsc.cumsum(zi) - zi
                s2 = _N_POS + tn + cf + plsc.cumsum(fi) - fi
                plsc.store_scatter(sa_v, [s1], lanes, mask=zm & (s1 < _N_POS))
                plsc.store_scatter(sa_v, [s2], lanes, mask=pm & (s2 < _S))
                return cz + jnp.sum(zi), cf + jnp.sum(fi)

            lax.fori_loop(0, NCH, fill_body, (i32(0), i32(0)))

            def pass3(k, _):
                base = k * 16
                s = sa_v[pl.ds(base, 16)]
                m = plsc.load_gather(mi_v, [s])
                hit = plsc.load_gather(mv_v, [s]) >= 0.5
                gl = plsc.load_gather(gl_v, [m])
                ol_v[pl.ds(base, 16)] = jnp.where(hit, gl, 0)
                for pv, ov in zip((p0, p1, p2, p3), (o0, o1, o2, o3)):
                    ov[pl.ds(base, 16)] = plsc.load_gather(pv, [s])
                for gv, qv in zip((g0, g1, g2, g3), (q0, q1, q2, q3)):
                    qv[pl.ds(base, 16)] = plsc.load_gather(gv, [m])
                return 0

            lax.fori_loop(0, _S // 16, pass3, 0)

            for c, ov in enumerate((o0, o1, o2, o3)):
                pltpu.sync_copy(ov, osp_h.at[img, c])
            for c, qv in enumerate((q0, q1, q2, q3)):
                pltpu.sync_copy(qv, ogt_h.at[img, c])
            pltpu.sync_copy(ol_v, olb_h.at[img])

    sc_call = pl.kernel(
        _sc_body,
        out_type=(
            jax.ShapeDtypeStruct((B, 4, _S), f32),
            jax.ShapeDtypeStruct((B, 4, _S), f32),
            jax.ShapeDtypeStruct((B, _S), i32),
        ),
        mesh=mesh,
        compiler_params=pltpu.CompilerParams(needs_layout_passes=False),
        scratch_types=[
            pltpu.VMEM((NPAD,), f32),
            pltpu.VMEM((NPAD,), i32),
            pltpu.VMEM((NPAD,), i32),
        ] + [pltpu.VMEM((NPAD,), f32)] * 4
        + [pltpu.VMEM((G,), f32)] * 4
        + [pltpu.VMEM((G,), i32), pltpu.VMEM((_S,), i32)]
        + [pltpu.VMEM((_S,), f32)] * 8
        + [pltpu.VMEM((_S,), i32)],
    )
    osp, ogt, olb = sc_call(mv, mi, perm_p, props_tp, gt_t,
                            gt_labels.astype(i32))

    reg4 = pl.pallas_call(
        _enc_body,
        grid=(B,),
        in_specs=[
            pl.BlockSpec((1, 4, _S), lambda i: (i, 0, 0)),
            pl.BlockSpec((1, 4, _S), lambda i: (i, 0, 0)),
        ],
        out_specs=pl.BlockSpec((1, 4, _S), lambda i: (i, 0, 0)),
        out_shape=jax.ShapeDtypeStruct((B, 4, _S), f32),
    )(osp, ogt)

    s_props = jnp.transpose(osp, (0, 2, 1))
    reg = jnp.transpose(reg4, (0, 2, 1))
    return (s_props, olb, reg)


# speedup vs baseline: 2.6509x; 1.0333x over previous
import functools

import jax
import jax.numpy as jnp
from jax import lax
from jax.experimental import pallas as pl
from jax.experimental.pallas import tpu as pltpu
from jax.experimental.pallas import tpu_sc as plsc

_N_POS = 128
_N_NEG = 384
_S = _N_POS + _N_NEG


def _match_body(G, p_ref, gt_ref, mv_ref, mi_ref):
    x1 = p_ref[0, 0]
    y1 = p_ref[0, 1]
    x2 = p_ref[0, 2]
    y2 = p_ref[0, 3]
    area_b = (x2 - x1) * (y2 - y1)
    best = jnp.full(x1.shape, -1.0, jnp.float32)
    bidx = jnp.zeros(x1.shape, jnp.int32)
    for g in range(G):
        ax1 = gt_ref[0, 0, g]
        ay1 = gt_ref[0, 1, g]
        ax2 = gt_ref[0, 2, g]
        ay2 = gt_ref[0, 3, g]
        area_a = (ax2 - ax1) * (ay2 - ay1)
        w = jnp.maximum(jnp.minimum(ax2, x2) - jnp.maximum(ax1, x1), 0.0)
        h = jnp.maximum(jnp.minimum(ay2, y2) - jnp.maximum(ay1, y1), 0.0)
        inter = w * h
        iou = inter / (area_a + area_b - inter + 1e-9)
        upd = iou > best
        best = jnp.where(upd, iou, best)
        bidx = jnp.where(upd, g, bidx)
    mv_ref[0] = best
    mi_ref[0] = bidx


def _enc_body(sp_ref, gt_ref, out_ref):
    sp = sp_ref[0]
    g = gt_ref[0]
    px1, py1, px2, py2 = sp[0:1], sp[1:2], sp[2:3], sp[3:4]
    gx1, gy1, gx2, gy2 = g[0:1], g[1:2], g[2:3], g[3:4]
    pw = px2 - px1
    ph = py2 - py1
    pxc = px1 + 0.5 * pw
    pyc = py1 + 0.5 * ph
    gw = gx2 - gx1
    gh = gy2 - gy1
    gxc = gx1 + 0.5 * gw
    gyc = gy1 + 0.5 * gh
    dx = 10.0 * (gxc - pxc) / pw
    dy = 10.0 * (gyc - pyc) / ph
    dw = 5.0 * jnp.log(gw / pw)
    dh = 5.0 * jnp.log(gh / ph)
    out_ref[0] = jnp.concatenate([dx, dy, dw, dh], axis=0)


def kernel(features, proposals, image_shapes, gt_boxes, gt_labels):
    del features, image_shapes
    i32 = jnp.int32
    f32 = jnp.float32
    B, N, _ = proposals.shape
    G = gt_boxes.shape[1]
    NP = N + G
    NPAD = -(-NP // 128) * 128
    NCH = -(-NP // 16)

    props_t = jnp.transpose(
        jnp.concatenate([proposals, gt_boxes], axis=1), (0, 2, 1)
    ).astype(f32)
    props_tp = jnp.pad(props_t, ((0, 0), (0, 0), (0, NPAD - NP)))
    gt_t = jnp.transpose(gt_boxes, (0, 2, 1)).astype(f32)

    perm = jnp.stack([
        jnp.argsort(
            -jax.random.uniform(jax.random.fold_in(jax.random.key(42), i), (NP,)),
            stable=True,
        ).astype(i32)
        for i in range(B)
    ])
    perm_p = jnp.pad(perm, ((0, 0), (0, NPAD - NP)))

    R = 8
    C = NPAD // R
    mv, mi = pl.pallas_call(
        functools.partial(_match_body, G),
        grid=(B,),
        in_specs=[
            pl.BlockSpec((1, 4, R, C), lambda i: (i, 0, 0, 0)),
            pl.BlockSpec((1, 4, G), lambda i: (i, 0, 0), memory_space=pltpu.SMEM),
        ],
        out_specs=[
            pl.BlockSpec((1, R, C), lambda i: (i, 0, 0)),
            pl.BlockSpec((1, R, C), lambda i: (i, 0, 0)),
        ],
        out_shape=[
            jax.ShapeDtypeStruct((B, R, C), f32),
            jax.ShapeDtypeStruct((B, R, C), i32),
        ],
    )(props_tp.reshape(B, 4, R, C), gt_t)
    mv = mv.reshape(B, NPAD)
    mi = mi.reshape(B, NPAD)

    info = plsc.get_sparse_core_info()
    NC = info.num_cores
    mesh = plsc.VectorSubcoreMesh(core_axis_name="c", subcore_axis_name="s")

    def _sc_body(mv_h, mi_h, pe_h, pr_h, gt_h, gl_h, osp_h, ogt_h, olb_h,
                 mv_v, mi_v, pe_v, p0, p1, p2, p3, g0, g1, g2, g3, gl_v,
                 sa_v, o0, o1, o2, o3, q0, q1, q2, q3, ol_v):
        wid = lax.axis_index("s") * NC + lax.axis_index("c")

        @pl.when(wid < B)
        def _():
            img = wid
            pltpu.sync_copy(mv_h.at[img], mv_v)
            pltpu.sync_copy(mi_h.at[img], mi_v)
            pltpu.sync_copy(pe_h.at[img], pe_v)
            for c, pv in enumerate((p0, p1, p2, p3)):
                pltpu.sync_copy(pr_h.at[img, c], pv)
            for c, gv in enumerate((g0, g1, g2, g3)):
                pltpu.sync_copy(gt_h.at[img, c], gv)
            pltpu.sync_copy(gl_h.at[img], gl_v)

            iota16 = lax.iota(i32, 16)

            def pass1(t, carry):
                def do_chunk(carry):
                    cp, cn = carry
                    base = t * 16
                    idx = pe_v[pl.ds(base, 16)]
                    valid = (base + iota16) < NP
                    hit = plsc.load_gather(mv_v, [idx]) >= 0.5
                    posm = hit & valid
                    negm = (~hit) & valid
                    pi = posm.astype(i32)
                    ni = negm.astype(i32)
                    gp = cp + plsc.cumsum(pi) - pi
                    gn = cn + plsc.cumsum(ni) - ni
                    plsc.store_scatter(sa_v, [gp], idx,
                                       mask=posm & (gp < _N_POS))
                    plsc.store_scatter(sa_v, [gn + _N_POS], idx,
                                       mask=negm & (gn < _N_NEG))
                    return cp + jnp.sum(pi), cn + jnp.sum(ni)

                cp, cn = carry
                return lax.cond((cp < _N_POS) | (cn < _N_NEG), do_chunk,
                                lambda c: c, carry)

            tp, tn = lax.fori_loop(0, NCH, pass1, (i32(0), i32(0)))

            @pl.when((tp < _N_POS) | (tn < _N_NEG))
            def _fill():
                def fill_body(t, carry):
                    cz, cf = carry
                    base = t * 16
                    lanes = base + iota16
                    valid = lanes < NP
                    hit = mv_v[pl.ds(base, 16)] >= 0.5
                    zm = (~hit) & valid
                    pm = hit & valid
                    zi = zm.astype(i32)
                    fi = pm.astype(i32)
                    s1 = tp + cz + plsc.cumsum(zi) - zi
                    s2 = _N_POS + tn + cf + plsc.cumsum(fi) - fi
                    plsc.store_scatter(sa_v, [s1], lanes,
                                       mask=zm & (s1 < _N_POS))
                    plsc.store_scatter(sa_v, [s2], lanes,
                                       mask=pm & (s2 < _S))
                    return cz + jnp.sum(zi), cf + jnp.sum(fi)

                lax.fori_loop(0, NCH, fill_body, (i32(0), i32(0)))

            def pass3(k, _):
                base = k * 16
                s = sa_v[pl.ds(base, 16)]
                m = plsc.load_gather(mi_v, [s])
                hit = plsc.load_gather(mv_v, [s]) >= 0.5
                gl = plsc.load_gather(gl_v, [m])
                ol_v[pl.ds(base, 16)] = jnp.where(hit, gl, 0)
                for pv, ov in zip((p0, p1, p2, p3), (o0, o1, o2, o3)):
                    ov[pl.ds(base, 16)] = plsc.load_gather(pv, [s])
                for gv, qv in zip((g0, g1, g2, g3), (q0, q1, q2, q3)):
                    qv[pl.ds(base, 16)] = plsc.load_gather(gv, [m])
                return 0

            lax.fori_loop(0, _S // 16, pass3, 0)

            for c, ov in enumerate((o0, o1, o2, o3)):
                pltpu.sync_copy(ov, osp_h.at[img, c])
            for c, qv in enumerate((q0, q1, q2, q3)):
                pltpu.sync_copy(qv, ogt_h.at[img, c])
            pltpu.sync_copy(ol_v, olb_h.at[img])

    sc_call = pl.kernel(
        _sc_body,
        out_type=(
            jax.ShapeDtypeStruct((B, 4, _S), f32),
            jax.ShapeDtypeStruct((B, 4, _S), f32),
            jax.ShapeDtypeStruct((B, _S), i32),
        ),
        mesh=mesh,
        compiler_params=pltpu.CompilerParams(needs_layout_passes=False),
        scratch_types=[
            pltpu.VMEM((NPAD,), f32),
            pltpu.VMEM((NPAD,), i32),
            pltpu.VMEM((NPAD,), i32),
        ] + [pltpu.VMEM((NPAD,), f32)] * 4
        + [pltpu.VMEM((G,), f32)] * 4
        + [pltpu.VMEM((G,), i32), pltpu.VMEM((_S,), i32)]
        + [pltpu.VMEM((_S,), f32)] * 8
        + [pltpu.VMEM((_S,), i32)],
    )
    osp, ogt, olb = sc_call(mv, mi, perm_p, props_tp, gt_t,
                            gt_labels.astype(i32))

    reg4 = pl.pallas_call(
        _enc_body,
        grid=(B,),
        in_specs=[
            pl.BlockSpec((1, 4, _S), lambda i: (i, 0, 0)),
            pl.BlockSpec((1, 4, _S), lambda i: (i, 0, 0)),
        ],
        out_specs=pl.BlockSpec((1, 4, _S), lambda i: (i, 0, 0)),
        out_shape=jax.ShapeDtypeStruct((B, 4, _S), f32),
    )(osp, ogt)

    s_props = jnp.transpose(osp, (0, 2, 1))
    reg = jnp.transpose(reg4, (0, 2, 1))
    return (s_props, olb, reg)


# speedup vs baseline: 2.7492x; 1.0371x over previous
import functools

import jax
import jax.numpy as jnp
from jax import lax
from jax.experimental import pallas as pl
from jax.experimental.pallas import tpu as pltpu
from jax.experimental.pallas import tpu_sc as plsc

_N_POS = 128
_N_NEG = 384
_S = _N_POS + _N_NEG
_LN2 = 0.6931471805599453


def _match_body(G, p_ref, gt_ref, out_ref):
    x1 = p_ref[0, 0]
    y1 = p_ref[0, 1]
    x2 = p_ref[0, 2]
    y2 = p_ref[0, 3]
    area_b = (x2 - x1) * (y2 - y1)
    best = jnp.full(x1.shape, -1.0, jnp.float32)
    bidx = jnp.zeros(x1.shape, jnp.int32)
    for g in range(G):
        ax1 = gt_ref[0, 0, g]
        ay1 = gt_ref[0, 1, g]
        ax2 = gt_ref[0, 2, g]
        ay2 = gt_ref[0, 3, g]
        area_a = (ax2 - ax1) * (ay2 - ay1)
        w = jnp.maximum(jnp.minimum(ax2, x2) - jnp.maximum(ax1, x1), 0.0)
        h = jnp.maximum(jnp.minimum(ay2, y2) - jnp.maximum(ay1, y1), 0.0)
        inter = w * h
        iou = inter / (area_a + area_b - inter + 1e-9)
        upd = iou > best
        best = jnp.where(upd, iou, best)
        bidx = jnp.where(upd, g, bidx)
    out_ref[0, 0] = lax.bitcast_convert_type(best, jnp.int32)
    out_ref[0, 1] = bidx


def _ln(t):
    i32 = jnp.int32
    f32 = jnp.float32
    bits = plsc.bitcast(t, i32)
    e = ((bits >> 23) - 127).astype(f32)
    m = plsc.bitcast((bits & 0x007FFFFF) | 0x3F800000, f32)
    z = (m - 1.0) / (m + 1.0)
    z2 = z * z
    p = 1.0 / 9.0
    p = p * z2 + 1.0 / 7.0
    p = p * z2 + 1.0 / 5.0
    p = p * z2 + 1.0 / 3.0
    p = p * z2 + 1.0
    return e * _LN2 + 2.0 * (z * p)


def kernel(features, proposals, image_shapes, gt_boxes, gt_labels):
    del features, image_shapes
    i32 = jnp.int32
    f32 = jnp.float32
    B, N, _ = proposals.shape
    G = gt_boxes.shape[1]
    NP = N + G
    NPAD = -(-NP // 128) * 128
    NCH = -(-NP // 16)

    props_tp = jnp.pad(
        jnp.transpose(jnp.concatenate([proposals, gt_boxes], axis=1),
                      (0, 2, 1)).astype(f32),
        ((0, 0), (0, 0), (0, NPAD - NP)))
    gt_t = jnp.transpose(gt_boxes, (0, 2, 1)).astype(f32)

    perm = jnp.stack([
        jnp.argsort(
            -jax.random.uniform(jax.random.fold_in(jax.random.key(42), i), (NP,)),
            stable=True,
        ).astype(i32)
        for i in range(B)
    ])
    perm_p = jnp.pad(perm, ((0, 0), (0, NPAD - NP)))

    R = 8
    C = NPAD // R
    mvmi = pl.pallas_call(
        functools.partial(_match_body, G),
        grid=(B,),
        in_specs=[
            pl.BlockSpec((1, 4, R, C), lambda i: (i, 0, 0, 0)),
            pl.BlockSpec((1, 4, G), lambda i: (i, 0, 0), memory_space=pltpu.SMEM),
        ],
        out_specs=pl.BlockSpec((1, 2, R, C), lambda i: (i, 0, 0, 0)),
        out_shape=jax.ShapeDtypeStruct((B, 2, R, C), i32),
    )(props_tp.reshape(B, 4, R, C), gt_t)
    mvmi = mvmi.reshape(B, 2 * NPAD)

    info = plsc.get_sparse_core_info()
    NC = info.num_cores
    mesh = plsc.VectorSubcoreMesh(core_axis_name="c", subcore_axis_name="s")

    def _sc_body(vmi_h, pe_h, pp_h, gt_h, osp_h, olb_h, org_h,
                 vmi_v, pe_v, pv, gv, sa_v, op_v, or_v, ol_v, sem):
        wid = lax.axis_index("s") * NC + lax.axis_index("c")

        @pl.when(wid < B)
        def _():
            img = wid
            h1 = pltpu.async_copy(vmi_h.at[img], vmi_v, sem)
            h2 = pltpu.async_copy(pe_h.at[img], pe_v, sem)
            h3 = pltpu.async_copy(pp_h.at[img], pv, sem)
            h4 = pltpu.async_copy(gt_h.at[img], gv, sem)
            h1.wait()
            h2.wait()

            iota16 = lax.iota(i32, 16)

            def pass1(t, carry):
                def do_chunk(carry):
                    cp, cn = carry
                    base = t * 16
                    idx = pe_v[pl.ds(base, 16)]
                    valid = (base + iota16) < NP
                    hit = plsc.bitcast(plsc.load_gather(vmi_v, [idx]), f32) >= 0.5
                    posm = hit & valid
                    negm = (~hit) & valid
                    pi = posm.astype(i32)
                    ni = negm.astype(i32)
                    gp = cp + plsc.cumsum(pi) - pi
                    gn = cn + plsc.cumsum(ni) - ni
                    plsc.store_scatter(sa_v, [gp], idx,
                                       mask=posm & (gp < _N_POS))
                    plsc.store_scatter(sa_v, [gn + _N_POS], idx,
                                       mask=negm & (gn < _N_NEG))
                    return cp + jnp.sum(pi), cn + jnp.sum(ni)

                cp, cn = carry
                return lax.cond((cp < _N_POS) | (cn < _N_NEG), do_chunk,
                                lambda c: c, carry)

            tp, tn = lax.fori_loop(0, NCH, pass1, (i32(0), i32(0)))

            @pl.when((tp < _N_POS) | (tn < _N_NEG))
            def _fill():
                def fill_body(t, carry):
                    cz, cf = carry
                    base = t * 16
                    lanes = base + iota16
                    valid = lanes < NP
                    hit = plsc.bitcast(vmi_v[pl.ds(base, 16)], f32) >= 0.5
                    zm = (~hit) & valid
                    pm = hit & valid
                    zi = zm.astype(i32)
                    fi = pm.astype(i32)
                    s1 = tp + cz + plsc.cumsum(zi) - zi
                    s2 = _N_POS + tn + cf + plsc.cumsum(fi) - fi
                    plsc.store_scatter(sa_v, [s1], lanes,
                                       mask=zm & (s1 < _N_POS))
                    plsc.store_scatter(sa_v, [s2], lanes,
                                       mask=pm & (s2 < _S))
                    return cz + jnp.sum(zi), cf + jnp.sum(fi)

                lax.fori_loop(0, NCH, fill_body, (i32(0), i32(0)))

            h3.wait()
            h4.wait()

            def pass3(k, _):
                base = k * 16
                rows4 = (base + iota16) * 4
                s = sa_v[pl.ds(base, 16)]
                m = plsc.load_gather(vmi_v, [NPAD + s])
                hit = plsc.bitcast(plsc.load_gather(vmi_v, [s]), f32) >= 0.5
                gl = plsc.bitcast(plsc.load_gather(gv, [4 * G + m]), i32)
                ol_v[pl.ds(base, 16)] = jnp.where(hit, gl, 0)
                pb = []
                gb = []
                for c in range(4):
                    pc = plsc.load_gather(pv, [c * NPAD + s])
                    gc = plsc.load_gather(gv, [c * G + m])
                    plsc.store_scatter(op_v, [rows4 + c], pc)
                    pb.append(pc)
                    gb.append(gc)
                px1, py1, px2, py2 = pb
                gx1, gy1, gx2, gy2 = gb
                pw = px2 - px1
                ph = py2 - py1
                pxc = px1 + 0.5 * pw
                pyc = py1 + 0.5 * ph
                gw = gx2 - gx1
                gh = gy2 - gy1
                gxc = gx1 + 0.5 * gw
                gyc = gy1 + 0.5 * gh
                enc = (10.0 * (gxc - pxc) / pw,
                       10.0 * (gyc - pyc) / ph,
                       5.0 * _ln(gw / pw),
                       5.0 * _ln(gh / ph))
                for c in range(4):
                    plsc.store_scatter(or_v, [rows4 + c], enc[c])
                return 0

            lax.fori_loop(0, _S // 16, pass3, 0)

            o1 = pltpu.async_copy(op_v, osp_h.at[img], sem)
            o2 = pltpu.async_copy(ol_v, olb_h.at[img], sem)
            o3 = pltpu.async_copy(or_v, org_h.at[img], sem)
            o1.wait()
            o2.wait()
            o3.wait()

    sc_call = pl.kernel(
        _sc_body,
        out_type=(
            jax.ShapeDtypeStruct((B, 4 * _S), f32),
            jax.ShapeDtypeStruct((B, _S), i32),
            jax.ShapeDtypeStruct((B, 4 * _S), f32),
        ),
        mesh=mesh,
        compiler_params=pltpu.CompilerParams(needs_layout_passes=False),
        scratch_types=[
            pltpu.VMEM((2 * NPAD,), i32),
            pltpu.VMEM((NPAD,), i32),
            pltpu.VMEM((4 * NPAD,), f32),
            pltpu.VMEM((5 * G,), f32),
            pltpu.VMEM((_S,), i32),
            pltpu.VMEM((4 * _S,), f32),
            pltpu.VMEM((4 * _S,), f32),
            pltpu.VMEM((_S,), i32),
            pltpu.SemaphoreType.DMA,
        ],
    )
    gt_f = jnp.concatenate([gt_t, lax.bitcast_convert_type(
        gt_labels.astype(i32), f32)[:, None]], axis=1)
    osp, olb, org = sc_call(mvmi, perm_p, props_tp.reshape(B, 4 * NPAD),
                            gt_f.reshape(B, 5 * G))
    return (osp.reshape(B, _S, 4), olb, org.reshape(B, _S, 4))


# speedup vs baseline: 2.7555x; 1.0023x over previous
import functools

import jax
import jax.numpy as jnp
from jax import lax
from jax.experimental import pallas as pl
from jax.experimental.pallas import tpu as pltpu
from jax.experimental.pallas import tpu_sc as plsc

_N_POS = 128
_N_NEG = 384
_S = _N_POS + _N_NEG
_LN2 = 0.6931471805599453


def _match_body(G, p_ref, gt_ref, out_ref):
    x1 = p_ref[0, 0]
    y1 = p_ref[0, 1]
    x2 = p_ref[0, 2]
    y2 = p_ref[0, 3]
    area_b = (x2 - x1) * (y2 - y1)
    best = jnp.full(x1.shape, -1.0, jnp.float32)
    bidx = jnp.zeros(x1.shape, jnp.int32)
    for g in range(G):
        ax1 = gt_ref[0, 0, g]
        ay1 = gt_ref[0, 1, g]
        ax2 = gt_ref[0, 2, g]
        ay2 = gt_ref[0, 3, g]
        area_a = (ax2 - ax1) * (ay2 - ay1)
        w = jnp.maximum(jnp.minimum(ax2, x2) - jnp.maximum(ax1, x1), 0.0)
        h = jnp.maximum(jnp.minimum(ay2, y2) - jnp.maximum(ay1, y1), 0.0)
        inter = w * h
        iou = inter / (area_a + area_b - inter + 1e-9)
        upd = iou > best
        best = jnp.where(upd, iou, best)
        bidx = jnp.where(upd, g, bidx)
    out_ref[0, 0] = lax.bitcast_convert_type(best, jnp.int32)
    out_ref[0, 1] = bidx


def _ln(t):
    i32 = jnp.int32
    f32 = jnp.float32
    bits = plsc.bitcast(t, i32)
    e = ((bits >> 23) - 127).astype(f32)
    m = plsc.bitcast((bits & 0x007FFFFF) | 0x3F800000, f32)
    z = (m - 1.0) / (m + 1.0)
    z2 = z * z
    p = 1.0 / 9.0
    p = p * z2 + 1.0 / 7.0
    p = p * z2 + 1.0 / 5.0
    p = p * z2 + 1.0 / 3.0
    p = p * z2 + 1.0
    return e * _LN2 + 2.0 * (z * p)


def kernel(features, proposals, image_shapes, gt_boxes, gt_labels):
    del features, image_shapes
    i32 = jnp.int32
    f32 = jnp.float32
    B, N, _ = proposals.shape
    G = gt_boxes.shape[1]
    NP = N + G
    NPAD = -(-NP // 128) * 128
    NCH = -(-NP // 16)

    props_tp = jnp.pad(
        jnp.transpose(jnp.concatenate([proposals, gt_boxes], axis=1),
                      (0, 2, 1)).astype(f32),
        ((0, 0), (0, 0), (0, NPAD - NP)))
    gt_t = jnp.transpose(gt_boxes, (0, 2, 1)).astype(f32)

    perm = jnp.stack([
        jnp.argsort(
            -jax.random.uniform(jax.random.fold_in(jax.random.key(42), i), (NP,)),
            stable=True,
        ).astype(i32)
        for i in range(B)
    ])
    perm_p = jnp.pad(perm, ((0, 0), (0, NPAD - NP)))

    R = 8
    C = NPAD // R
    mvmi = pl.pallas_call(
        functools.partial(_match_body, G),
        grid=(B,),
        in_specs=[
            pl.BlockSpec((1, 4, R, C), lambda i: (i, 0, 0, 0)),
            pl.BlockSpec((1, 4, G), lambda i: (i, 0, 0), memory_space=pltpu.SMEM),
        ],
        out_specs=pl.BlockSpec((1, 2, R, C), lambda i: (i, 0, 0, 0)),
        out_shape=jax.ShapeDtypeStruct((B, 2, R, C), i32),
    )(props_tp.reshape(B, 4, R, C), gt_t)
    mvmi = mvmi.reshape(B, 2 * NPAD)

    info = plsc.get_sparse_core_info()
    NC = info.num_cores
    mesh = plsc.VectorSubcoreMesh(core_axis_name="c", subcore_axis_name="s")

    def _sc_body(vmi_h, pe_h, pp_h, gt_h, osp_h, olb_h, org_h,
                 vmi_v, pe_v, pv, gv, sa_v, op_v, or_v, ol_v, sem):
        wid = lax.axis_index("s") * NC + lax.axis_index("c")

        @pl.when(wid < B)
        def _():
            img = wid
            h1 = pltpu.async_copy(vmi_h.at[img], vmi_v, sem)
            h2 = pltpu.async_copy(pe_h.at[img], pe_v, sem)
            h3 = pltpu.async_copy(pp_h.at[img], pv, sem)
            h4 = pltpu.async_copy(gt_h.at[img], gv, sem)
            h1.wait()
            h2.wait()

            iota16 = lax.iota(i32, 16)

            def pass1(t, carry):
                def do_chunk(carry):
                    cp, cn = carry
                    base = t * 16
                    idx = pe_v[pl.ds(base, 16)]
                    valid = (base + iota16) < NP
                    hit = plsc.bitcast(plsc.load_gather(vmi_v, [idx]), f32) >= 0.5
                    posm = hit & valid
                    negm = (~hit) & valid
                    pi = posm.astype(i32)
                    ni = negm.astype(i32)
                    gp = cp + plsc.cumsum(pi) - pi
                    gn = cn + plsc.cumsum(ni) - ni
                    plsc.store_scatter(sa_v, [gp], idx,
                                       mask=posm & (gp < _N_POS))
                    plsc.store_scatter(sa_v, [gn + _N_POS], idx,
                                       mask=negm & (gn < _N_NEG))
                    return cp + jnp.sum(pi), cn + jnp.sum(ni)

                cp, cn = carry
                return lax.cond((cp < _N_POS) | (cn < _N_NEG), do_chunk,
                                lambda c: c, carry)

            tp, tn = lax.fori_loop(0, NCH, pass1, (i32(0), i32(0)))

            @pl.when((tp < _N_POS) | (tn < _N_NEG))
            def _fill():
                def fill_body(t, carry):
                    cz, cf = carry
                    base = t * 16
                    lanes = base + iota16
                    valid = lanes < NP
                    hit = plsc.bitcast(vmi_v[pl.ds(base, 16)], f32) >= 0.5
                    zm = (~hit) & valid
                    pm = hit & valid
                    zi = zm.astype(i32)
                    fi = pm.astype(i32)
                    s1 = tp + cz + plsc.cumsum(zi) - zi
                    s2 = _N_POS + tn + cf + plsc.cumsum(fi) - fi
                    plsc.store_scatter(sa_v, [s1], lanes,
                                       mask=zm & (s1 < _N_POS))
                    plsc.store_scatter(sa_v, [s2], lanes,
                                       mask=pm & (s2 < _S))
                    return cz + jnp.sum(zi), cf + jnp.sum(fi)

                lax.fori_loop(0, NCH, fill_body, (i32(0), i32(0)))

            h3.wait()
            h4.wait()

            def pass3(k, _):
                base = k * 16
                rows4 = (base + iota16) * 4
                s = sa_v[pl.ds(base, 16)]
                m = plsc.load_gather(vmi_v, [NPAD + s])
                hit = plsc.bitcast(plsc.load_gather(vmi_v, [s]), f32) >= 0.5
                gl = plsc.load_gather(gv, [4 * G + m])
                ol_v[pl.ds(base, 16)] = jnp.where(hit, gl, 0)
                pb = []
                gb = []
                for c in range(4):
                    pc = plsc.load_gather(pv, [c * NPAD + s])
                    gc = plsc.bitcast(plsc.load_gather(gv, [c * G + m]), f32)
                    plsc.store_scatter(op_v, [rows4 + c], pc)
                    pb.append(pc)
                    gb.append(gc)
                px1, py1, px2, py2 = pb
                gx1, gy1, gx2, gy2 = gb
                pw = px2 - px1
                ph = py2 - py1
                pxc = px1 + 0.5 * pw
                pyc = py1 + 0.5 * ph
                gw = gx2 - gx1
                gh = gy2 - gy1
                gxc = gx1 + 0.5 * gw
                gyc = gy1 + 0.5 * gh
                enc = (10.0 * (gxc - pxc) / pw,
                       10.0 * (gyc - pyc) / ph,
                       5.0 * _ln(gw / pw),
                       5.0 * _ln(gh / ph))
                for c in range(4):
                    plsc.store_scatter(or_v, [rows4 + c], enc[c])
                return 0

            lax.fori_loop(0, _S // 16, pass3, 0)

            o1 = pltpu.async_copy(op_v, osp_h.at[img], sem)
            o2 = pltpu.async_copy(ol_v, olb_h.at[img], sem)
            o3 = pltpu.async_copy(or_v, org_h.at[img], sem)
            o1.wait()
            o2.wait()
            o3.wait()

    sc_call = pl.kernel(
        _sc_body,
        out_type=(
            jax.ShapeDtypeStruct((B, 4 * _S), f32),
            jax.ShapeDtypeStruct((B, _S), i32),
            jax.ShapeDtypeStruct((B, 4 * _S), f32),
        ),
        mesh=mesh,
        compiler_params=pltpu.CompilerParams(needs_layout_passes=False),
        scratch_types=[
            pltpu.VMEM((2 * NPAD,), i32),
            pltpu.VMEM((NPAD,), i32),
            pltpu.VMEM((4 * NPAD,), f32),
            pltpu.VMEM((5 * G,), i32),
            pltpu.VMEM((_S,), i32),
            pltpu.VMEM((4 * _S,), f32),
            pltpu.VMEM((4 * _S,), f32),
            pltpu.VMEM((_S,), i32),
            pltpu.SemaphoreType.DMA,
        ],
    )
    gt_f = jnp.concatenate(
        [lax.bitcast_convert_type(gt_t, i32),
         gt_labels.astype(i32)[:, None]], axis=1)
    osp, olb, org = sc_call(mvmi, perm_p, props_tp.reshape(B, 4 * NPAD),
                            gt_f.reshape(B, 5 * G))
    return (osp.reshape(B, _S, 4), olb, org.reshape(B, _S, 4))


# speedup vs baseline: 4.0089x; 1.4549x over previous
import functools

import jax
import jax.numpy as jnp
from jax import lax
from jax.experimental import pallas as pl
from jax.experimental.pallas import tpu as pltpu
from jax.experimental.pallas import tpu_sc as plsc

_N_POS = 128
_N_NEG = 384
_S = _N_POS + _N_NEG
_LN2 = 0.6931471805599453


def _match_body(G, p_ref, gt_ref, out_ref):
    x1 = p_ref[0, 0]
    y1 = p_ref[0, 1]
    x2 = p_ref[0, 2]
    y2 = p_ref[0, 3]
    area_b = (x2 - x1) * (y2 - y1)
    best = jnp.full(x1.shape, -1.0, jnp.float32)
    bidx = jnp.zeros(x1.shape, jnp.int32)
    for g in range(G):
        ax1 = gt_ref[0, 0, g]
        ay1 = gt_ref[0, 1, g]
        ax2 = gt_ref[0, 2, g]
        ay2 = gt_ref[0, 3, g]
        area_a = (ax2 - ax1) * (ay2 - ay1)
        w = jnp.maximum(jnp.minimum(ax2, x2) - jnp.maximum(ax1, x1), 0.0)
        h = jnp.maximum(jnp.minimum(ay2, y2) - jnp.maximum(ay1, y1), 0.0)
        inter = w * h
        iou = inter / (area_a + area_b - inter + 1e-9)
        upd = iou > best
        best = jnp.where(upd, iou, best)
        bidx = jnp.where(upd, g, bidx)
    out_ref[0, 0] = lax.bitcast_convert_type(best, jnp.int32)
    out_ref[0, 1] = bidx


def _ln(t):
    i32 = jnp.int32
    f32 = jnp.float32
    bits = plsc.bitcast(t, i32)
    e = ((bits >> 23) - 127).astype(f32)
    m = plsc.bitcast((bits & 0x007FFFFF) | 0x3F800000, f32)
    z = (m - 1.0) / (m + 1.0)
    z2 = z * z
    p = 1.0 / 9.0
    p = p * z2 + 1.0 / 7.0
    p = p * z2 + 1.0 / 5.0
    p = p * z2 + 1.0 / 3.0
    p = p * z2 + 1.0
    return e * _LN2 + 2.0 * (z * p)


def kernel(features, proposals, image_shapes, gt_boxes, gt_labels):
    del features, image_shapes
    i32 = jnp.int32
    f32 = jnp.float32
    B, N, _ = proposals.shape
    G = gt_boxes.shape[1]
    NP = N + G
    NPAD = -(-NP // 128) * 128
    NCH = -(-NP // 16)

    props_tp = jnp.pad(
        jnp.transpose(jnp.concatenate([proposals, gt_boxes], axis=1),
                      (0, 2, 1)).astype(f32),
        ((0, 0), (0, 0), (0, NPAD - NP)))
    gt_t = jnp.transpose(gt_boxes, (0, 2, 1)).astype(f32)

    with jax.ensure_compile_time_eval():
        perm = jnp.stack([
            jnp.argsort(
                -jax.random.uniform(
                    jax.random.fold_in(jax.random.key(42), i), (NP,)),
                stable=True,
            ).astype(i32)
            for i in range(B)
        ])
        perm_p = jnp.pad(perm, ((0, 0), (0, NPAD - NP)))

    R = 8
    C = NPAD // R
    mvmi = pl.pallas_call(
        functools.partial(_match_body, G),
        grid=(B,),
        in_specs=[
            pl.BlockSpec((1, 4, R, C), lambda i: (i, 0, 0, 0)),
            pl.BlockSpec((1, 4, G), lambda i: (i, 0, 0), memory_space=pltpu.SMEM),
        ],
        out_specs=pl.BlockSpec((1, 2, R, C), lambda i: (i, 0, 0, 0)),
        out_shape=jax.ShapeDtypeStruct((B, 2, R, C), i32),
    )(props_tp.reshape(B, 4, R, C), gt_t)
    mvmi = mvmi.reshape(B, 2 * NPAD)

    info = plsc.get_sparse_core_info()
    NC = info.num_cores
    mesh = plsc.VectorSubcoreMesh(core_axis_name="c", subcore_axis_name="s")

    def _sc_body(vmi_h, pe_h, pp_h, gt_h, osp_h, olb_h, org_h,
                 vmi_v, pe_v, pv, gv, sa_v, op_v, or_v, ol_v, sem):
        wid = lax.axis_index("s") * NC + lax.axis_index("c")

        @pl.when(wid < B)
        def _():
            img = wid
            h1 = pltpu.async_copy(vmi_h.at[img], vmi_v, sem)
            h2 = pltpu.async_copy(pe_h.at[img], pe_v, sem)
            h3 = pltpu.async_copy(pp_h.at[img], pv, sem)
            h4 = pltpu.async_copy(gt_h.at[img], gv, sem)
            h1.wait()
            h2.wait()

            iota16 = lax.iota(i32, 16)

            def pass1(t, carry):
                def do_chunk(carry):
                    cp, cn = carry
                    base = t * 16
                    idx = pe_v[pl.ds(base, 16)]
                    valid = (base + iota16) < NP
                    hit = plsc.bitcast(plsc.load_gather(vmi_v, [idx]), f32) >= 0.5
                    posm = hit & valid
                    negm = (~hit) & valid
                    pi = posm.astype(i32)
                    ni = negm.astype(i32)
                    gp = cp + plsc.cumsum(pi) - pi
                    gn = cn + plsc.cumsum(ni) - ni
                    plsc.store_scatter(sa_v, [gp], idx,
                                       mask=posm & (gp < _N_POS))
                    plsc.store_scatter(sa_v, [gn + _N_POS], idx,
                                       mask=negm & (gn < _N_NEG))
                    return cp + jnp.sum(pi), cn + jnp.sum(ni)

                cp, cn = carry
                return lax.cond((cp < _N_POS) | (cn < _N_NEG), do_chunk,
                                lambda c: c, carry)

            tp, tn = lax.fori_loop(0, NCH, pass1, (i32(0), i32(0)))

            @pl.when((tp < _N_POS) | (tn < _N_NEG))
            def _fill():
                def fill_body(t, carry):
                    cz, cf = carry
                    base = t * 16
                    lanes = base + iota16
                    valid = lanes < NP
                    hit = plsc.bitcast(vmi_v[pl.ds(base, 16)], f32) >= 0.5
                    zm = (~hit) & valid
                    pm = hit & valid
                    zi = zm.astype(i32)
                    fi = pm.astype(i32)
                    s1 = tp + cz + plsc.cumsum(zi) - zi
                    s2 = _N_POS + tn + cf + plsc.cumsum(fi) - fi
                    plsc.store_scatter(sa_v, [s1], lanes,
                                       mask=zm & (s1 < _N_POS))
                    plsc.store_scatter(sa_v, [s2], lanes,
                                       mask=pm & (s2 < _S))
                    return cz + jnp.sum(zi), cf + jnp.sum(fi)

                lax.fori_loop(0, NCH, fill_body, (i32(0), i32(0)))

            h3.wait()
            h4.wait()

            def pass3(k, _):
                base = k * 16
                rows4 = (base + iota16) * 4
                s = sa_v[pl.ds(base, 16)]
                m = plsc.load_gather(vmi_v, [NPAD + s])
                hit = plsc.bitcast(plsc.load_gather(vmi_v, [s]), f32) >= 0.5
                gl = plsc.load_gather(gv, [4 * G + m])
                ol_v[pl.ds(base, 16)] = jnp.where(hit, gl, 0)
                pb = []
                gb = []
                for c in range(4):
                    pc = plsc.load_gather(pv, [c * NPAD + s])
                    gc = plsc.bitcast(plsc.load_gather(gv, [c * G + m]), f32)
                    plsc.store_scatter(op_v, [rows4 + c], pc)
                    pb.append(pc)
                    gb.append(gc)
                px1, py1, px2, py2 = pb
                gx1, gy1, gx2, gy2 = gb
                pw = px2 - px1
                ph = py2 - py1
                pxc = px1 + 0.5 * pw
                pyc = py1 + 0.5 * ph
                gw = gx2 - gx1
                gh = gy2 - gy1
                gxc = gx1 + 0.5 * gw
                gyc = gy1 + 0.5 * gh
                enc = (10.0 * (gxc - pxc) / pw,
                       10.0 * (gyc - pyc) / ph,
                       5.0 * _ln(gw / pw),
                       5.0 * _ln(gh / ph))
                for c in range(4):
                    plsc.store_scatter(or_v, [rows4 + c], enc[c])
                return 0

            lax.fori_loop(0, _S // 16, pass3, 0)

            o1 = pltpu.async_copy(op_v, osp_h.at[img], sem)
            o2 = pltpu.async_copy(ol_v, olb_h.at[img], sem)
            o3 = pltpu.async_copy(or_v, org_h.at[img], sem)
            o1.wait()
            o2.wait()
            o3.wait()

    sc_call = pl.kernel(
        _sc_body,
        out_type=(
            jax.ShapeDtypeStruct((B, 4 * _S), f32),
            jax.ShapeDtypeStruct((B, _S), i32),
            jax.ShapeDtypeStruct((B, 4 * _S), f32),
        ),
        mesh=mesh,
        compiler_params=pltpu.CompilerParams(needs_layout_passes=False),
        scratch_types=[
            pltpu.VMEM((2 * NPAD,), i32),
            pltpu.VMEM((NPAD,), i32),
            pltpu.VMEM((4 * NPAD,), f32),
            pltpu.VMEM((5 * G,), i32),
            pltpu.VMEM((_S,), i32),
            pltpu.VMEM((4 * _S,), f32),
            pltpu.VMEM((4 * _S,), f32),
            pltpu.VMEM((_S,), i32),
            pltpu.SemaphoreType.DMA,
        ],
    )
    gt_f = jnp.concatenate(
        [lax.bitcast_convert_type(gt_t, i32),
         gt_labels.astype(i32)[:, None]], axis=1)
    osp, olb, org = sc_call(mvmi, perm_p, props_tp.reshape(B, 4 * NPAD),
                            gt_f.reshape(B, 5 * G))
    return (osp.reshape(B, _S, 4), olb, org.reshape(B, _S, 4))


# speedup vs baseline: 4.7125x; 1.1755x over previous
import functools

import jax
import jax.numpy as jnp
from jax import lax
from jax.experimental import pallas as pl
from jax.experimental.pallas import tpu as pltpu
from jax.experimental.pallas import tpu_sc as plsc

_N_POS = 128
_N_NEG = 384
_S = _N_POS + _N_NEG
_LN2 = 0.6931471805599453


def _match_body(G, p_ref, gt_ref, out_ref):
    x1 = p_ref[0, 0]
    y1 = p_ref[0, 1]
    x2 = p_ref[0, 2]
    y2 = p_ref[0, 3]
    area_b = (x2 - x1) * (y2 - y1)
    best = jnp.full(x1.shape, -1.0, jnp.float32)
    bidx = jnp.zeros(x1.shape, jnp.int32)
    for g in range(G):
        ax1 = gt_ref[0, 0, g]
        ay1 = gt_ref[0, 1, g]
        ax2 = gt_ref[0, 2, g]
        ay2 = gt_ref[0, 3, g]
        area_a = (ax2 - ax1) * (ay2 - ay1)
        w = jnp.maximum(jnp.minimum(ax2, x2) - jnp.maximum(ax1, x1), 0.0)
        h = jnp.maximum(jnp.minimum(ay2, y2) - jnp.maximum(ay1, y1), 0.0)
        inter = w * h
        iou = inter / (area_a + area_b - inter + 1e-9)
        upd = iou > best
        best = jnp.where(upd, iou, best)
        bidx = jnp.where(upd, g, bidx)
    out_ref[0, 0] = lax.bitcast_convert_type(best, jnp.int32)
    out_ref[0, 1] = bidx


def _ln(t):
    i32 = jnp.int32
    f32 = jnp.float32
    bits = plsc.bitcast(t, i32)
    e = ((bits >> 23) - 127).astype(f32)
    m = plsc.bitcast((bits & 0x007FFFFF) | 0x3F800000, f32)
    z = (m - 1.0) / (m + 1.0)
    z2 = z * z
    p = 1.0 / 9.0
    p = p * z2 + 1.0 / 7.0
    p = p * z2 + 1.0 / 5.0
    p = p * z2 + 1.0 / 3.0
    p = p * z2 + 1.0
    return e * _LN2 + 2.0 * (z * p)


def kernel(features, proposals, image_shapes, gt_boxes, gt_labels):
    del features, image_shapes
    i32 = jnp.int32
    f32 = jnp.float32
    B, N, _ = proposals.shape
    G = gt_boxes.shape[1]
    NP = N + G
    NPAD = -(-NP // 128) * 128
    NCH = -(-NP // 16)

    props_tp = jnp.pad(
        jnp.transpose(jnp.concatenate([proposals, gt_boxes], axis=1),
                      (0, 2, 1)).astype(f32),
        ((0, 0), (0, 0), (0, NPAD - NP)))
    gt_t = jnp.transpose(gt_boxes, (0, 2, 1)).astype(f32)

    def _build_perm():
        perm = jnp.stack([
            jnp.argsort(
                -jax.random.uniform(
                    jax.random.fold_in(jax.random.key(42), i), (NP,)),
                stable=True,
            ).astype(i32)
            for i in range(B)
        ])
        return jnp.pad(perm, ((0, 0), (0, NPAD - NP)))

    try:
        with jax.ensure_compile_time_eval():
            perm_p = _build_perm()
    except Exception:
        perm_p = _build_perm()

    R = 8
    C = NPAD // R
    mvmi = pl.pallas_call(
        functools.partial(_match_body, G),
        grid=(B,),
        in_specs=[
            pl.BlockSpec((1, 4, R, C), lambda i: (i, 0, 0, 0)),
            pl.BlockSpec((1, 4, G), lambda i: (i, 0, 0), memory_space=pltpu.SMEM),
        ],
        out_specs=pl.BlockSpec((1, 2, R, C), lambda i: (i, 0, 0, 0)),
        out_shape=jax.ShapeDtypeStruct((B, 2, R, C), i32),
    )(props_tp.reshape(B, 4, R, C), gt_t)
    mvmi = mvmi.reshape(B, 2 * NPAD)

    info = plsc.get_sparse_core_info()
    NC = info.num_cores
    mesh = plsc.VectorSubcoreMesh(core_axis_name="c", subcore_axis_name="s")

    def _sc_body(vmi_h, pe_h, pp_h, gt_h, osp_h, olb_h, org_h,
                 vmi_v, pe_v, pv, gv, sa_v, op_v, or_v, ol_v, sem):
        wid = lax.axis_index("s") * NC + lax.axis_index("c")

        @pl.when(wid < B)
        def _():
            img = wid
            h1 = pltpu.async_copy(vmi_h.at[img], vmi_v, sem)
            h2 = pltpu.async_copy(pe_h.at[img], pe_v, sem)
            h3 = pltpu.async_copy(pp_h.at[img], pv, sem)
            h4 = pltpu.async_copy(gt_h.at[img], gv, sem)
            h1.wait()
            h2.wait()

            iota16 = lax.iota(i32, 16)

            def chunk16(base, cp, cn):
                idx = pe_v[pl.ds(base, 16)]
                valid = (base + iota16) < NP
                hit = plsc.bitcast(plsc.load_gather(vmi_v, [idx]), f32) >= 0.5
                posm = hit & valid
                negm = (~hit) & valid
                pi = posm.astype(i32)
                ni = negm.astype(i32)
                csp = plsc.cumsum(pi)
                csn = plsc.cumsum(ni)
                gp = cp + csp - pi
                gn = cn + csn - ni
                plsc.store_scatter(sa_v, [gp], idx, mask=posm & (gp < _N_POS))
                plsc.store_scatter(sa_v, [gn + _N_POS], idx,
                                   mask=negm & (gn < _N_NEG))
                return cp + csp[15], cn + csn[15]

            UNROLL = 4
            NSUP = -(-NCH // UNROLL)

            def pass1(t, carry):
                def do_super(carry):
                    cp, cn = carry
                    for j in range(UNROLL):
                        cp, cn = chunk16((t * UNROLL + j) * 16, cp, cn)
                    return cp, cn

                cp, cn = carry
                return lax.cond((cp < _N_POS) | (cn < _N_NEG), do_super,
                                lambda c: c, carry)

            tp, tn = lax.fori_loop(0, NSUP, pass1, (i32(0), i32(0)))

            @pl.when((tp < _N_POS) | (tn < _N_NEG))
            def _fill():
                def fill_body(t, carry):
                    cz, cf = carry
                    base = t * 16
                    lanes = base + iota16
                    valid = lanes < NP
                    hit = plsc.bitcast(vmi_v[pl.ds(base, 16)], f32) >= 0.5
                    zm = (~hit) & valid
                    pm = hit & valid
                    zi = zm.astype(i32)
                    fi = pm.astype(i32)
                    csz = plsc.cumsum(zi)
                    csf = plsc.cumsum(fi)
                    s1 = tp + cz + csz - zi
                    s2 = _N_POS + tn + cf + csf - fi
                    plsc.store_scatter(sa_v, [s1], lanes,
                                       mask=zm & (s1 < _N_POS))
                    plsc.store_scatter(sa_v, [s2], lanes,
                                       mask=pm & (s2 < _S))
                    return cz + csz[15], cf + csf[15]

                lax.fori_loop(0, NCH, fill_body, (i32(0), i32(0)))

            h3.wait()
            h4.wait()

            def pass3(k, _):
                base = k * 16
                rows4 = (base + iota16) * 4
                s = sa_v[pl.ds(base, 16)]
                m = plsc.load_gather(vmi_v, [NPAD + s])
                hit = plsc.bitcast(plsc.load_gather(vmi_v, [s]), f32) >= 0.5
                gl = plsc.load_gather(gv, [4 * G + m])
                ol_v[pl.ds(base, 16)] = jnp.where(hit, gl, 0)
                pb = []
                gb = []
                for c in range(4):
                    pc = plsc.load_gather(pv, [c * NPAD + s])
                    gc = plsc.bitcast(plsc.load_gather(gv, [c * G + m]), f32)
                    plsc.store_scatter(op_v, [rows4 + c], pc)
                    pb.append(pc)
                    gb.append(gc)
                px1, py1, px2, py2 = pb
                gx1, gy1, gx2, gy2 = gb
                pw = px2 - px1
                ph = py2 - py1
                pxc = px1 + 0.5 * pw
                pyc = py1 + 0.5 * ph
                gw = gx2 - gx1
                gh = gy2 - gy1
                gxc = gx1 + 0.5 * gw
                gyc = gy1 + 0.5 * gh
                enc = (10.0 * (gxc - pxc) / pw,
                       10.0 * (gyc - pyc) / ph,
                       5.0 * _ln(gw / pw),
                       5.0 * _ln(gh / ph))
                for c in range(4):
                    plsc.store_scatter(or_v, [rows4 + c], enc[c])
                return 0

            lax.fori_loop(0, _S // 16, pass3, 0, unroll=4)

            o1 = pltpu.async_copy(op_v, osp_h.at[img], sem)
            o2 = pltpu.async_copy(ol_v, olb_h.at[img], sem)
            o3 = pltpu.async_copy(or_v, org_h.at[img], sem)
            o1.wait()
            o2.wait()
            o3.wait()

    sc_call = pl.kernel(
        _sc_body,
        out_type=(
            jax.ShapeDtypeStruct((B, 4 * _S), f32),
            jax.ShapeDtypeStruct((B, _S), i32),
            jax.ShapeDtypeStruct((B, 4 * _S), f32),
        ),
        mesh=mesh,
        compiler_params=pltpu.CompilerParams(needs_layout_passes=False),
        scratch_types=[
            pltpu.VMEM((2 * NPAD,), i32),
            pltpu.VMEM((NPAD,), i32),
            pltpu.VMEM((4 * NPAD,), f32),
            pltpu.VMEM((5 * G,), i32),
            pltpu.VMEM((_S,), i32),
            pltpu.VMEM((4 * _S,), f32),
            pltpu.VMEM((4 * _S,), f32),
            pltpu.VMEM((_S,), i32),
            pltpu.SemaphoreType.DMA,
        ],
    )
    gt_f = jnp.concatenate(
        [lax.bitcast_convert_type(gt_t, i32),
         gt_labels.astype(i32)[:, None]], axis=1)
    osp, olb, org = sc_call(mvmi, perm_p, props_tp.reshape(B, 4 * NPAD),
                            gt_f.reshape(B, 5 * G))
    return (osp.reshape(B, _S, 4), olb, org.reshape(B, _S, 4))


# speedup vs baseline: 4.7561x; 1.0092x over previous
import functools

import jax
import jax.numpy as jnp
from jax import lax
from jax.experimental import pallas as pl
from jax.experimental.pallas import tpu as pltpu
from jax.experimental.pallas import tpu_sc as plsc

_N_POS = 128
_N_NEG = 384
_S = _N_POS + _N_NEG
_LN2 = 0.6931471805599453


def _match_body(G, p_ref, gt_ref, out_ref):
    x1 = p_ref[0, 0]
    y1 = p_ref[0, 1]
    x2 = p_ref[0, 2]
    y2 = p_ref[0, 3]
    area_b = (x2 - x1) * (y2 - y1)
    best = jnp.full(x1.shape, -1.0, jnp.float32)
    bidx = jnp.zeros(x1.shape, jnp.int32)
    for g in range(G):
        ax1 = gt_ref[0, 0, g]
        ay1 = gt_ref[0, 1, g]
        ax2 = gt_ref[0, 2, g]
        ay2 = gt_ref[0, 3, g]
        area_a = (ax2 - ax1) * (ay2 - ay1)
        w = jnp.maximum(jnp.minimum(ax2, x2) - jnp.maximum(ax1, x1), 0.0)
        h = jnp.maximum(jnp.minimum(ay2, y2) - jnp.maximum(ay1, y1), 0.0)
        inter = w * h
        iou = inter / (area_a + area_b - inter + 1e-9)
        upd = iou > best
        best = jnp.where(upd, iou, best)
        bidx = jnp.where(upd, g, bidx)
    out_ref[0, 0] = lax.bitcast_convert_type(best, jnp.int32)
    out_ref[0, 1] = bidx


def _ln(t):
    i32 = jnp.int32
    f32 = jnp.float32
    bits = plsc.bitcast(t, i32)
    e = ((bits >> 23) - 127).astype(f32)
    m = plsc.bitcast((bits & 0x007FFFFF) | 0x3F800000, f32)
    z = (m - 1.0) / (m + 1.0)
    z2 = z * z
    p = 1.0 / 9.0
    p = p * z2 + 1.0 / 7.0
    p = p * z2 + 1.0 / 5.0
    p = p * z2 + 1.0 / 3.0
    p = p * z2 + 1.0
    return e * _LN2 + 2.0 * (z * p)


def kernel(features, proposals, image_shapes, gt_boxes, gt_labels):
    del features, image_shapes
    i32 = jnp.int32
    f32 = jnp.float32
    B, N, _ = proposals.shape
    G = gt_boxes.shape[1]
    NP = N + G
    NPAD = -(-NP // 128) * 128
    NCH = -(-NP // 16)

    props_tp = jnp.pad(
        jnp.transpose(jnp.concatenate([proposals, gt_boxes], axis=1),
                      (0, 2, 1)).astype(f32),
        ((0, 0), (0, 0), (0, NPAD - NP)))
    gt_t = jnp.transpose(gt_boxes, (0, 2, 1)).astype(f32)

    def _build_perm():
        perm = jnp.stack([
            jnp.argsort(
                -jax.random.uniform(
                    jax.random.fold_in(jax.random.key(42), i), (NP,)),
                stable=True,
            ).astype(i32)
            for i in range(B)
        ])
        return jnp.pad(perm, ((0, 0), (0, NPAD - NP)))

    try:
        with jax.ensure_compile_time_eval():
            perm_p = _build_perm()
    except Exception:
        perm_p = _build_perm()

    R = 8
    C = NPAD // R
    mvmi = pl.pallas_call(
        functools.partial(_match_body, G),
        grid=(B,),
        in_specs=[
            pl.BlockSpec((1, 4, R, C), lambda i: (i, 0, 0, 0)),
            pl.BlockSpec((1, 4, G), lambda i: (i, 0, 0), memory_space=pltpu.SMEM),
        ],
        out_specs=pl.BlockSpec((1, 2, R, C), lambda i: (i, 0, 0, 0)),
        out_shape=jax.ShapeDtypeStruct((B, 2, R, C), i32),
    )(props_tp.reshape(B, 4, R, C), gt_t)
    mvmi = mvmi.reshape(B, 2 * NPAD)

    info = plsc.get_sparse_core_info()
    NC = info.num_cores
    mesh = plsc.VectorSubcoreMesh(core_axis_name="c", subcore_axis_name="s")

    def _sc_body(vmi_h, pe_h, pp_h, gt_h, osp_h, olb_h, org_h,
                 vmi_v, pe_v, pv, gv, sa_v, op_v, or_v, ol_v, sem):
        wid = lax.axis_index("s") * NC + lax.axis_index("c")

        @pl.when(wid < B)
        def _():
            img = wid
            h1 = pltpu.async_copy(vmi_h.at[img], vmi_v, sem)
            h2 = pltpu.async_copy(pe_h.at[img], pe_v, sem)
            h3 = pltpu.async_copy(pp_h.at[img], pv, sem)
            h4 = pltpu.async_copy(gt_h.at[img], gv, sem)
            h1.wait()
            h2.wait()

            iota16 = lax.iota(i32, 16)

            def chunk16(base, cp, cn):
                idx = pe_v[pl.ds(base, 16)]
                valid = (base + iota16) < NP
                hit = plsc.bitcast(plsc.load_gather(vmi_v, [idx]), f32) >= 0.5
                posm = hit & valid
                negm = (~hit) & valid
                pi = posm.astype(i32)
                ni = negm.astype(i32)
                csp = plsc.cumsum(pi)
                csn = plsc.cumsum(ni)
                gp = cp + csp - pi
                gn = cn + csn - ni
                plsc.store_scatter(sa_v, [gp], idx, mask=posm & (gp < _N_POS))
                plsc.store_scatter(sa_v, [gn + _N_POS], idx,
                                   mask=negm & (gn < _N_NEG))
                return cp + csp[15], cn + csn[15]

            UNROLL = 8
            NSUP = -(-NCH // UNROLL)

            def pass1(t, carry):
                def do_super(carry):
                    cp, cn = carry
                    for j in range(UNROLL):
                        cp, cn = chunk16((t * UNROLL + j) * 16, cp, cn)
                    return cp, cn

                cp, cn = carry
                return lax.cond((cp < _N_POS) | (cn < _N_NEG), do_super,
                                lambda c: c, carry)

            tp, tn = lax.fori_loop(0, NSUP, pass1, (i32(0), i32(0)))

            @pl.when((tp < _N_POS) | (tn < _N_NEG))
            def _fill():
                def fill_body(t, carry):
                    cz, cf = carry
                    base = t * 16
                    lanes = base + iota16
                    valid = lanes < NP
                    hit = plsc.bitcast(vmi_v[pl.ds(base, 16)], f32) >= 0.5
                    zm = (~hit) & valid
                    pm = hit & valid
                    zi = zm.astype(i32)
                    fi = pm.astype(i32)
                    csz = plsc.cumsum(zi)
                    csf = plsc.cumsum(fi)
                    s1 = tp + cz + csz - zi
                    s2 = _N_POS + tn + cf + csf - fi
                    plsc.store_scatter(sa_v, [s1], lanes,
                                       mask=zm & (s1 < _N_POS))
                    plsc.store_scatter(sa_v, [s2], lanes,
                                       mask=pm & (s2 < _S))
                    return cz + csz[15], cf + csf[15]

                lax.fori_loop(0, NCH, fill_body, (i32(0), i32(0)))

            h3.wait()
            h4.wait()

            def pass3(k, _):
                base = k * 16
                rows4 = (base + iota16) * 4
                s = sa_v[pl.ds(base, 16)]
                m = plsc.load_gather(vmi_v, [NPAD + s])
                hit = plsc.bitcast(plsc.load_gather(vmi_v, [s]), f32) >= 0.5
                gl = plsc.load_gather(gv, [4 * G + m])
                ol_v[pl.ds(base, 16)] = jnp.where(hit, gl, 0)
                pb = []
                gb = []
                for c in range(4):
                    pc = plsc.load_gather(pv, [c * NPAD + s])
                    gc = plsc.bitcast(plsc.load_gather(gv, [c * G + m]), f32)
                    plsc.store_scatter(op_v, [rows4 + c], pc)
                    pb.append(pc)
                    gb.append(gc)
                px1, py1, px2, py2 = pb
                gx1, gy1, gx2, gy2 = gb
                pw = px2 - px1
                ph = py2 - py1
                pxc = px1 + 0.5 * pw
                pyc = py1 + 0.5 * ph
                gw = gx2 - gx1
                gh = gy2 - gy1
                gxc = gx1 + 0.5 * gw
                gyc = gy1 + 0.5 * gh
                enc = (10.0 * (gxc - pxc) / pw,
                       10.0 * (gyc - pyc) / ph,
                       5.0 * _ln(gw / pw),
                       5.0 * _ln(gh / ph))
                for c in range(4):
                    plsc.store_scatter(or_v, [rows4 + c], enc[c])
                return 0

            lax.fori_loop(0, _S // 16, pass3, 0, unroll=4)

            o1 = pltpu.async_copy(op_v, osp_h.at[img], sem)
            o2 = pltpu.async_copy(ol_v, olb_h.at[img], sem)
            o3 = pltpu.async_copy(or_v, org_h.at[img], sem)
            o1.wait()
            o2.wait()
            o3.wait()

    sc_call = pl.kernel(
        _sc_body,
        out_type=(
            jax.ShapeDtypeStruct((B, 4 * _S), f32),
            jax.ShapeDtypeStruct((B, _S), i32),
            jax.ShapeDtypeStruct((B, 4 * _S), f32),
        ),
        mesh=mesh,
        compiler_params=pltpu.CompilerParams(needs_layout_passes=False),
        scratch_types=[
            pltpu.VMEM((2 * NPAD,), i32),
            pltpu.VMEM((NPAD,), i32),
            pltpu.VMEM((4 * NPAD,), f32),
            pltpu.VMEM((5 * G,), i32),
            pltpu.VMEM((_S,), i32),
            pltpu.VMEM((4 * _S,), f32),
            pltpu.VMEM((4 * _S,), f32),
            pltpu.VMEM((_S,), i32),
            pltpu.SemaphoreType.DMA,
        ],
    )
    gt_f = jnp.concatenate(
        [lax.bitcast_convert_type(gt_t, i32),
         gt_labels.astype(i32)[:, None]], axis=1)
    osp, olb, org = sc_call(mvmi, perm_p, props_tp.reshape(B, 4 * NPAD),
                            gt_f.reshape(B, 5 * G))
    return (osp.reshape(B, _S, 4), olb, org.reshape(B, _S, 4))
